# baseline (speedup 1.0000x reference)
import functools

import jax
import jax.numpy as jnp
from jax import lax
from jax.experimental import pallas as pl
from jax.experimental.pallas import tpu as pltpu
from jax.experimental.pallas import tpu_sc as plsc


B, S, D = 16, 4096, 512
T = 4096
L = 16


def _tc_body(mask_ref, x_ref, out_ref):
    b = pl.program_id(0)
    row = mask_ref[pl.ds(b, 1), :]
    len_b = jnp.sum(row).astype(jnp.int32)
    idx = lax.broadcasted_iota(jnp.int32, (T, 1), 0)
    valid = (idx < len_b).astype(x_ref.dtype)
    out_ref[0] = x_ref[0] * valid


def _sc_body(x2_hbm, mask_hbm, final_hbm, rest_hbm,
             mall_v, idx_v, rows_v, rest_v, sem):
    c = lax.axis_index("c")
    s = lax.axis_index("s")

    @pl.when((c == 0) & (s == 0))
    def _():
        pltpu.sync_copy(mask_hbm, mall_v)
        iota = lax.iota(jnp.int32, L)
        dn = lax.GatherDimensionNumbers(
            offset_dims=(), collapsed_slice_dims=(0,), start_index_map=(0,))

        lens = jnp.zeros((L,), jnp.int32)
        for r in range(B):
            def chunk(k, acc, r=r):
                return acc + mall_v[r, pl.ds(k * L, L)]

            acc = lax.fori_loop(0, S // L, chunk, jnp.zeros((L,), jnp.int32),
                                unroll=4)
            for sh in (8, 4, 2, 1):
                idx = ((iota + sh) & (L - 1)).reshape(L, 1)
                acc = acc + lax.gather(acc, idx, dn, (1,),
                                       mode=lax.GatherScatterMode.PROMISE_IN_BOUNDS)
            lens = jnp.where(iota == r, acc, lens)

        idx_v[...] = iota * S + lens - 1
        pltpu.async_copy(x2_hbm.at[idx_v], rows_v, sem).wait()
        pltpu.sync_copy(rows_v, final_hbm)

        key = lens * L + (L - 1 - iota)
        _, vals = plsc.sort_key_val(key, iota, descending=True)
        plsc.store_scatter(rest_v, [vals], iota)
        pltpu.sync_copy(rest_v, rest_hbm)


_sc_kernel = functools.partial(
    pl.kernel,
    out_type=[
        jax.ShapeDtypeStruct((B, D), jnp.float32),
        jax.ShapeDtypeStruct((B,), jnp.int32),
    ],
    mesh=plsc.VectorSubcoreMesh(core_axis_name="c", subcore_axis_name="s", num_cores=1),
    compiler_params=pltpu.CompilerParams(needs_layout_passes=False),
    scratch_types=[
        pltpu.VMEM((B, S), jnp.int32),
        pltpu.VMEM((L,), jnp.int32),
        pltpu.VMEM((B, D), jnp.float32),
        pltpu.VMEM((B,), jnp.int32),
        pltpu.SemaphoreType.DMA,
    ],
)(_sc_body)


@jax.jit
def kernel(inputs, mask):
    mask_f = mask.astype(jnp.float32)
    mask_i = mask.astype(jnp.int32)
    final, rest = _sc_kernel(inputs.reshape(B * S, D), mask_i)
    outputs = pl.pallas_call(
        _tc_body,
        grid=(B,),
        in_specs=[
            pl.BlockSpec((B, S), lambda b: (0, 0)),
            pl.BlockSpec((1, T, D), lambda b: (b, 0, 0)),
        ],
        out_specs=pl.BlockSpec((1, T, D), lambda b: (b, 0, 0)),
        out_shape=jax.ShapeDtypeStruct((B, S, D), inputs.dtype),
    )(mask_f, inputs)
    return outputs, final, rest


# speedup vs baseline: 1.0026x; 1.0026x over previous
import functools

import jax
import jax.numpy as jnp
from jax import lax
from jax.experimental import pallas as pl
from jax.experimental.pallas import tpu as pltpu
from jax.experimental.pallas import tpu_sc as plsc


B, S, D = 16, 4096, 512
T = 4096
L = 16


def _tc_body(mask_ref, x_ref, out_ref):
    b = pl.program_id(0)
    row = mask_ref[pl.ds(b, 1), :]
    len_b = jnp.sum(row).astype(jnp.int32)
    idx = lax.broadcasted_iota(jnp.int32, (T, 1), 0)
    valid = (idx < len_b).astype(x_ref.dtype)
    out_ref[0] = x_ref[0] * valid


def _sc_body(x2_hbm, mask_hbm, final_hbm, rest_hbm,
             mall_v, idx_v, rows_v, rest_v, sem):
    c = lax.axis_index("c")
    s = lax.axis_index("s")

    @pl.when((c == 0) & (s == 0))
    def _():
        pltpu.sync_copy(mask_hbm, mall_v)
        iota = lax.iota(jnp.int32, L)
        dn = lax.GatherDimensionNumbers(
            offset_dims=(), collapsed_slice_dims=(0,), start_index_map=(0,))

        lens = jnp.zeros((L,), jnp.int32)
        for r in range(B):
            def chunk(k, acc, r=r):
                return acc + mall_v[r, pl.ds(k * L, L)]

            acc = lax.fori_loop(0, S // L, chunk, jnp.zeros((L,), jnp.int32),
                                unroll=4)
            for sh in (8, 4, 2, 1):
                idx = ((iota + sh) & (L - 1)).reshape(L, 1)
                acc = acc + lax.gather(acc, idx, dn, (1,),
                                       mode=lax.GatherScatterMode.PROMISE_IN_BOUNDS)
            lens = jnp.where(iota == r, acc, lens)

        idx_v[...] = iota * S + lens - 1
        pltpu.async_copy(x2_hbm.at[idx_v], rows_v, sem).wait()
        pltpu.sync_copy(rows_v, final_hbm)

        key = lens * L + (L - 1 - iota)
        _, vals = plsc.sort_key_val(key, iota, descending=True)
        plsc.store_scatter(rest_v, [vals], iota)
        pltpu.sync_copy(rest_v, rest_hbm)


_sc_kernel = functools.partial(
    pl.kernel,
    out_type=[
        jax.ShapeDtypeStruct((B, D), jnp.float32),
        jax.ShapeDtypeStruct((B,), jnp.int32),
    ],
    mesh=plsc.VectorSubcoreMesh(core_axis_name="c", subcore_axis_name="s", num_cores=1, num_subcores=1),
    compiler_params=pltpu.CompilerParams(needs_layout_passes=False),
    scratch_types=[
        pltpu.VMEM((B, S), jnp.int32),
        pltpu.VMEM((L,), jnp.int32),
        pltpu.VMEM((B, D), jnp.float32),
        pltpu.VMEM((B,), jnp.int32),
        pltpu.SemaphoreType.DMA,
    ],
)(_sc_body)


@jax.jit
def kernel(inputs, mask):
    mask_f = mask.astype(jnp.float32)
    mask_i = mask.astype(jnp.int32)
    final, rest = _sc_kernel(inputs.reshape(B * S, D), mask_i)
    outputs = pl.pallas_call(
        _tc_body,
        grid=(B,),
        in_specs=[
            pl.BlockSpec((B, S), lambda b: (0, 0)),
            pl.BlockSpec((1, T, D), lambda b: (b, 0, 0)),
        ],
        out_specs=pl.BlockSpec((1, T, D), lambda b: (b, 0, 0)),
        out_shape=jax.ShapeDtypeStruct((B, S, D), inputs.dtype),
    )(mask_f, inputs)
    return outputs, final, rest
